# Initial kernel scaffold; baseline (speedup 1.0000x reference)
import jax
import jax.numpy as jnp
from jax.experimental import pallas as pl


def kernel(path, obstacles, edge_index, loop, params):
    raise NotImplementedError("write your pallas kernel here")


# speedup vs baseline: 2.1334x; 2.1334x over previous
import functools

import jax
import jax.numpy as jnp
from jax import lax
from jax.experimental import pallas as pl
from jax.experimental.pallas import tpu as pltpu
from jax.experimental.pallas import tpu_sc as plsc

EMBED = 64
CFG = 7
OBS = 6
N_NODES = 10000
N_OBST = 32
N_EDGES = 160000

H = 2 * EMBED

NB = 2000
EB = 3200

_TEMP = float(EMBED) ** 0.5


def _ln(x, g, b):
    mu = jnp.mean(x, axis=-1, keepdims=True)
    var = jnp.mean((x - mu) ** 2, axis=-1, keepdims=True)
    return (x - mu) * lax.rsqrt(var + 1e-6) * g + b


def _dotT(x, w):
    return lax.dot_general(x, w, (((1,), (1,)), ((), ())),
                           preferred_element_type=jnp.float32)


def _prelude_body(path_ref, obst_ref,
                  wnc_ref, bnc_ref,
                  nfw1_ref, nfb1_ref, nfw2_ref, nfb2_ref,
                  onw1_ref, onb1_ref, onw2_ref, onb2_ref,
                  attw_ref, attln_ref,
                  mfw_ref, mfb_ref, mfln_ref,
                  ofw_ref, ofb_ref, ofln_ref,
                  wea1_ref, wea2_ref, be_ref,
                  nc_ref, nf_ref, p0_ref):
    path = path_ref[...]
    obst = obst_ref[...]

    nc = _dotT(path, wnc_ref[...]) + bnc_ref[...]
    nf = _dotT(jax.nn.relu(_dotT(path, nfw1_ref[...]) + nfb1_ref[...]),
               nfw2_ref[...]) + nfb2_ref[...]
    oc = _dotT(jax.nn.relu(_dotT(obst, onw1_ref[...]) + onb1_ref[...]),
               onw2_ref[...]) + onb2_ref[...]

    def ff(x, w, bb, lnp):
        h = _dotT(jax.nn.relu(_dotT(x, w[0]) + bb[0]), w[1]) + bb[1]
        return _ln(x + h, lnp[0], lnp[1])

    for blk in range(3):
        aw = attw_ref[blk]
        aln = attln_ref[blk]
        mq = _dotT(nf, aw[0])
        mk = _dotT(nf, aw[1])
        mv = _dotT(nf, aw[2])
        ok = _dotT(oc, aw[1])
        ov = _dotT(oc, aw[2])
        l0 = jnp.sum(mq * mk, axis=-1, keepdims=True) / _TEMP
        lo = _dotT(mq, ok) / _TEMP
        mx = jnp.maximum(l0, jnp.max(lo, axis=-1, keepdims=True))
        e0 = jnp.exp(l0 - mx)
        eo = jnp.exp(lo - mx)
        z = e0 + jnp.sum(eo, axis=-1, keepdims=True)
        new = (e0 / z) * mv + jnp.dot(eo / z, ov,
                                      preferred_element_type=jnp.float32)
        nf = _ln(new + nf, aln[0], aln[1])
        nf = ff(nf, mfw_ref[blk], mfb_ref[blk], mfln_ref[blk])
        oc = ff(oc, ofw_ref[blk], ofb_ref[blk], ofln_ref[blk])

    p0 = _dotT(nc, wea1_ref[...]) + _dotT(nf, wea2_ref[...]) + be_ref[...]
    nc_ref[...] = nc
    nf_ref[...] = nf
    p0_ref[...] = p0


def _k1_body(p0_ref, hi_ref, web_ref, wsrc_ref, b1_ref, wdst_ref,
             enc_ref, u_ref, v_ref):
    enc = p0_ref[...] + _dotT(hi_ref[...], web_ref[...])
    enc_ref[...] = enc
    u_ref[...] = _dotT(enc, wsrc_ref[...]) + b1_ref[...]
    v_ref[...] = _dotT(enc, wdst_ref[...])


def _k2_body(m_ref, w2_ref, b2c_ref, yt_ref):
    yt_ref[...] = lax.dot_general(
        w2_ref[...], jax.nn.relu(m_ref[...]), (((1,), (1,)), ((), ())),
        preferred_element_type=jnp.float32) + b2c_ref[...]


def _k3_body(enc_ref, outt_ref, nc_ref, l1a_ref, l1b_ref, lb_ref,
             da_ref, dbm_ref, db_ref, hi_ref, dec_ref):
    outt = jnp.maximum(outt_ref[0], outt_ref[1])
    outt = jnp.where(outt == -jnp.inf, 0.0, outt)
    hi = (_dotT(enc_ref[...], l1a_ref[...])
          + lax.dot_general(outt, l1b_ref[...],
                            (((0,), (1,)), ((), ())),
                            preferred_element_type=jnp.float32)
          + lb_ref[...])
    hi_ref[...] = hi
    dec_ref[...] = (_dotT(nc_ref[...], da_ref[...])
                    + _dotT(hi, dbm_ref[...]) + db_ref[...])


def _k4_body(dec_ref, sw_ref, sb_ref, res_ref):
    res_ref[...] = _dotT(dec_ref[...], sw_ref[...]) + sb_ref[...]


def _full(shape):
    return pl.BlockSpec(shape, lambda *_: tuple(0 for _ in shape))


def _rows(shape):
    n = len(shape)
    return pl.BlockSpec(shape, lambda i: (i,) + tuple(0 for _ in range(n - 1)))


_SC_NC, _SC_NS = 2, 16
_NW = _SC_NC * _SC_NS
_EPT = N_EDGES // _NW
_GC = 200

_CW = H // _SC_NS
_EHALF = N_EDGES // _SC_NC
_SCH = 3200
_NSCH = _EHALF // _SCH
_NGR = _SCH // 16

_mesh = plsc.VectorSubcoreMesh(core_axis_name="c", subcore_axis_name="s")
_sc_params = pltpu.CompilerParams(needs_layout_passes=False)


def _wid():
    return lax.axis_index("s") * _SC_NC + lax.axis_index("c")


_GH = (96, 104)


def _sc_gather_body(u_hbm, v_hbm, src_hbm, dst_hbm, m_hbm,
                    sidx0, didx0, ubuf0, vbuf0,
                    sidx1, didx1, ubuf1, vbuf1,
                    obuf0, obuf1, gsem0, gsem1, wsem0, wsem1):
    base = _wid() * _EPT
    nch = _EPT // _GC
    bufs = ((sidx0, didx0, ubuf0, vbuf0, gsem0),
            (sidx1, didx1, ubuf1, vbuf1, gsem1))
    obufs = ((obuf0, wsem0), (obuf1, wsem1))

    def start(k, b):
        e0 = base + k * _GC
        sidx, didx, ubuf, vbuf, gsem = bufs[b]
        pltpu.sync_copy(src_hbm.at[pl.ds(e0, _GC)], sidx)
        pltpu.sync_copy(dst_hbm.at[pl.ds(e0, _GC)], didx)
        pltpu.async_copy(u_hbm.at[sidx], ubuf, gsem)
        pltpu.async_copy(v_hbm.at[didx], vbuf, gsem)

    def finish(k, b, drain_wb):
        e0 = base + k * _GC
        sidx, didx, ubuf, vbuf, gsem = bufs[b]
        pltpu.make_async_copy(u_hbm.at[sidx], ubuf, gsem).wait()
        pltpu.make_async_copy(v_hbm.at[didx], vbuf, gsem).wait()
        r0 = 0
        for h in range(2):
            obuf, wsem = obufs[h]
            gh = _GH[h]

            @pl.when(drain_wb)
            def _():
                pltpu.make_async_copy(
                    obuf, m_hbm.at[pl.ds(e0, gh)], wsem).wait()

            def row(i, c2, r0=r0):
                for j in range(H // 16):
                    sl = pl.ds(j * 16, 16)
                    obuf[i, sl] = ubuf[r0 + i, sl] + vbuf[r0 + i, sl]
                return c2

            lax.fori_loop(0, gh, row, 0)
            pltpu.async_copy(obuf, m_hbm.at[pl.ds(e0 + r0, gh)], wsem)
            r0 += gh

    start(0, 0)

    def step(k, carry):
        b = lax.rem(k, 2)

        @pl.when(k + 1 < nch)
        def _():
            lax.switch(1 - b, [lambda: start(k + 1, 0),
                               lambda: start(k + 1, 1)])
        lax.switch(b, [lambda: finish(k, 0, k >= 1),
                       lambda: finish(k, 1, k >= 1)])
        return carry

    lax.fori_loop(0, nch, step, 0)
    for h in range(2):
        obuf, wsem = obufs[h]
        pltpu.make_async_copy(obuf, m_hbm.at[pl.ds(base, _GH[h])],
                              wsem).wait()


_sc_gather = pl.kernel(
    _sc_gather_body,
    out_type=jax.ShapeDtypeStruct((N_EDGES, H), jnp.float32),
    mesh=_mesh,
    compiler_params=_sc_params,
    scratch_types=[
        pltpu.VMEM((_GC,), jnp.int32),
        pltpu.VMEM((_GC,), jnp.int32),
        pltpu.VMEM((_GC, H), jnp.float32),
        pltpu.VMEM((_GC, H), jnp.float32),
        pltpu.VMEM((_GC,), jnp.int32),
        pltpu.VMEM((_GC,), jnp.int32),
        pltpu.VMEM((_GC, H), jnp.float32),
        pltpu.VMEM((_GC, H), jnp.float32),
        pltpu.VMEM((_GH[0], H), jnp.float32),
        pltpu.VMEM((_GH[1], H), jnp.float32),
        pltpu.SemaphoreType.DMA,
        pltpu.SemaphoreType.DMA,
        pltpu.SemaphoreType.DMA,
        pltpu.SemaphoreType.DMA,
    ],
)


def _sc_scatter_body(yt_hbm, dst_hbm, outt_hbm, acc, ybuf, dbuf, sbuf, sem):
    half = lax.axis_index("c")
    r0 = lax.axis_index("s") * _CW
    ebase = half * _EHALF
    neg = jnp.full((16,), -jnp.inf, jnp.float32)
    iota = lax.iota(jnp.int32, 16)
    mask8 = iota < _CW

    for c in range(_CW):
        def initrow(i, c2, c=c):
            acc[c, pl.ds(i * 16, 16)] = neg
            return c2
        lax.fori_loop(0, N_NODES // 16, initrow, 0)

    def chunk(k, carry):
        e0 = ebase + k * _SCH
        pltpu.sync_copy(dst_hbm.at[pl.ds(e0, _SCH)], dbuf)
        cp = pltpu.async_copy(yt_hbm.at[pl.ds(r0, _CW), pl.ds(e0, _SCH)],
                              ybuf, sem)
        cp.wait()

        def group(g, c2):
            off = g * 16
            d16 = dbuf[pl.ds(off, 16)]
            plsc.store_scatter(sbuf, [d16], iota)
            rb = plsc.load_gather(sbuf, [d16])
            anydup = jnp.any(rb != iota)

            def fast(_):
                for c in range(_CW):
                    yv = ybuf[c, pl.ds(off, 16)]
                    cvec = jnp.full((16,), c, jnp.int32)
                    av = plsc.load_gather(acc, [cvec, d16])
                    plsc.store_scatter(acc, [cvec, d16], jnp.maximum(av, yv))
                return 0

            def slow(_):
                iotac = jnp.where(mask8, iota, 0)

                def per_edge(j, c3):
                    col = jnp.full((16,), 1, jnp.int32) * (off + j)
                    dj = plsc.load_gather(dbuf, [col], mask=mask8)
                    dj = jnp.where(mask8, dj, 0)
                    yv = plsc.load_gather(ybuf, [iotac, col], mask=mask8)
                    av = plsc.load_gather(acc, [iotac, dj], mask=mask8)
                    plsc.store_scatter(acc, [iotac, dj], jnp.maximum(av, yv),
                                       mask=mask8)
                    return c3
                lax.fori_loop(0, 16, per_edge, 0)
                return 0

            lax.cond(anydup, slow, fast, 0)
            return c2

        lax.fori_loop(0, _NGR, group, 0)
        return carry

    lax.fori_loop(0, _NSCH, chunk, 0)

    pltpu.sync_copy(acc, outt_hbm.at[half, pl.ds(r0, _CW), :])


_sc_scatter = pl.kernel(
    _sc_scatter_body,
    out_type=jax.ShapeDtypeStruct((_SC_NC, H, N_NODES), jnp.float32),
    mesh=_mesh,
    compiler_params=_sc_params,
    scratch_types=[
        pltpu.VMEM((_CW, N_NODES), jnp.float32),
        pltpu.VMEM((_CW, _SCH), jnp.float32),
        pltpu.VMEM((_SCH,), jnp.int32),
        pltpu.VMEM((N_NODES,), jnp.int32),
        pltpu.SemaphoreType.DMA,
    ],
)


def kernel(path, obstacles, edge_index, loop, params):
    f32 = jnp.float32
    grid_n = N_NODES // NB

    path_p = jnp.pad(path, ((0, 0), (0, 1)))
    obst_p = jnp.pad(obstacles, ((0, 0), (0, 2)))
    wnc = jnp.pad(params['node_code']['W'], ((0, 0), (0, 1)))
    bnc = params['node_code']['b'].reshape(1, EMBED)
    nfw1 = jnp.pad(params['nfc']['W1'], ((0, 0), (0, 1)))
    nfb1 = params['nfc']['b1'].reshape(1, EMBED)
    nfw2 = params['nfc']['W2']
    nfb2 = params['nfc']['b2'].reshape(1, EMBED)
    onw1 = jnp.pad(params['onc']['W1'], ((0, 0), (0, 2)))
    onb1 = params['onc']['b1'].reshape(1, EMBED)
    onw2 = params['onc']['W2']
    onb2 = params['onc']['b2'].reshape(1, EMBED)

    blocks = params['blocks']
    attw = jnp.stack([jnp.stack([bp['attn']['Wq'], bp['attn']['Wk'],
                                 bp['attn']['Wv']]) for bp in blocks])
    attln = jnp.stack([jnp.stack([bp['attn']['ln_g'], bp['attn']['ln_b']])
                       for bp in blocks])

    def ffpack(key):
        w = jnp.stack([jnp.stack([bp[key]['W1'], bp[key]['W2']])
                       for bp in blocks])
        b = jnp.stack([jnp.stack([bp[key]['b1'], bp[key]['b2']])
                       for bp in blocks])
        g = jnp.stack([jnp.stack([bp[key]['ln_g'], bp[key]['ln_b']])
                       for bp in blocks])
        return w, b, g

    mfw, mfb, mfln = ffpack('map_ff')
    ofw, ofb, ofln = ffpack('obs_ff')

    we = params['encoder']['W']
    wea1, wea2, web = we[:, :EMBED], we[:, EMBED:H], we[:, H:]
    be = params['encoder']['b'].reshape(1, H)

    w1 = params['mpnn']['lin0_W1']
    wsrc = w1[:, :H] + w1[:, H:2 * H]
    wdst = w1[:, 2 * H:] - w1[:, :H]
    b1 = params['mpnn']['lin0_b1'].reshape(1, H)
    w2 = params['mpnn']['lin0_W2']
    b2c = params['mpnn']['lin0_b2'].reshape(H, 1)
    l1 = params['mpnn']['lin1_W']
    l1a, l1b = l1[:, :H], l1[:, H:]
    lb = params['mpnn']['lin1_b'].reshape(1, H)
    dw = params['decoder']['W']
    da, dbm = dw[:, :EMBED], dw[:, EMBED:]
    db = params['decoder']['b'].reshape(1, EMBED)
    sw = jnp.pad(params['smooth']['W'], ((0, 1), (0, 0)))
    sb = jnp.pad(params['smooth']['b'], (0, 1)).reshape(1, 8)

    src = edge_index[0]
    dst = edge_index[1]

    nc, nf, p0 = pl.pallas_call(
        _prelude_body,
        grid=(grid_n,),
        in_specs=[
            _rows((NB, 8)), _full((N_OBST, 8)),
            _full(wnc.shape), _full(bnc.shape),
            _full(nfw1.shape), _full(nfb1.shape),
            _full(nfw2.shape), _full(nfb2.shape),
            _full(onw1.shape), _full(onb1.shape),
            _full(onw2.shape), _full(onb2.shape),
            _full(attw.shape), _full(attln.shape),
            _full(mfw.shape), _full(mfb.shape), _full(mfln.shape),
            _full(ofw.shape), _full(ofb.shape), _full(ofln.shape),
            _full(wea1.shape), _full(wea2.shape), _full(be.shape),
        ],
        out_specs=[_rows((NB, EMBED)), _rows((NB, EMBED)), _rows((NB, H))],
        out_shape=[jax.ShapeDtypeStruct((N_NODES, EMBED), f32),
                   jax.ShapeDtypeStruct((N_NODES, EMBED), f32),
                   jax.ShapeDtypeStruct((N_NODES, H), f32)],
    )(path_p, obst_p, wnc, bnc, nfw1, nfb1, nfw2, nfb2,
      onw1, onb1, onw2, onb2, attw, attln, mfw, mfb, mfln,
      ofw, ofb, ofln, wea1, wea2, be)

    h0 = jnp.concatenate([nc, nf], axis=1)

    k1 = pl.pallas_call(
        _k1_body,
        grid=(grid_n,),
        in_specs=[_rows((NB, H)), _rows((NB, H)), _full((H, H)),
                  _full((H, H)), _full((1, H)), _full((H, H))],
        out_specs=[_rows((NB, H))] * 3,
        out_shape=[jax.ShapeDtypeStruct((N_NODES, H), f32)] * 3,
    )

    k2 = pl.pallas_call(
        _k2_body,
        grid=(N_EDGES // EB,),
        in_specs=[_rows((EB, H)), _full((H, H)), _full((H, 1))],
        out_specs=[pl.BlockSpec((H, EB), lambda j: (0, j))],
        out_shape=[jax.ShapeDtypeStruct((H, N_EDGES), f32)],
    )

    k3 = pl.pallas_call(
        _k3_body,
        grid=(1,),
        in_specs=[_full((N_NODES, H)), _full((_SC_NC, H, N_NODES)),
                  _full((N_NODES, EMBED)), _full((H, H)), _full((H, H)),
                  _full((1, H)), _full((EMBED, EMBED)), _full((EMBED, H)),
                  _full((1, EMBED))],
        out_specs=[_full((N_NODES, H)), _full((N_NODES, EMBED))],
        out_shape=[jax.ShapeDtypeStruct((N_NODES, H), f32),
                   jax.ShapeDtypeStruct((N_NODES, EMBED), f32)],
    )

    def body(_, carry):
        hi, dec = carry
        enc, u, v = k1(p0, hi, web, wsrc, b1, wdst)
        m = _sc_gather(u, v, src, dst)
        yt, = k2(m, w2, b2c)
        outt = _sc_scatter(yt, dst)
        hi2, dec2 = k3(enc, outt, nc, l1a, l1b, lb, da, dbm, db)
        return hi2, dec2

    hi, dec = lax.fori_loop(0, loop, body,
                            (h0, jnp.zeros((N_NODES, EMBED), f32)))

    res8, = pl.pallas_call(
        _k4_body,
        grid=(grid_n,),
        in_specs=[_rows((NB, EMBED)), _full((8, EMBED)), _full((1, 8))],
        out_specs=[_rows((NB, 8))],
        out_shape=[jax.ShapeDtypeStruct((N_NODES, 8), f32)],
    )(dec, sw, sb)
    return res8[:, :CFG]
